# baseline (speedup 1.0000x reference)
import functools

import jax
import jax.numpy as jnp
from jax import lax
from jax.experimental import pallas as pl
from jax.experimental.pallas import tpu as pltpu
from jax.experimental.pallas import tpu_sc as plsc

_NUM_CORES = 2
_NUM_SUBCORES = 16
_NUM_WORKERS = _NUM_CORES * _NUM_SUBCORES

_CB = 256
_CC = 384
_L = 16


def _mesh():
    return plsc.VectorSubcoreMesh(
        core_axis_name="c", subcore_axis_name="s",
        num_cores=_NUM_CORES, num_subcores=_NUM_SUBCORES)


@functools.lru_cache(maxsize=None)
def _make_repack(vocab: int, dim: int):
    assert dim == 64 and _CC % 128 == 0
    n_full = vocab // _CC
    tail = vocab - n_full * _CC
    assert tail % 8 == 0
    n_units = n_full + (1 if tail else 0)
    units_per_w = -(-n_units // _NUM_WORKERS)
    rows_per_u = _CC // 2

    @functools.partial(
        pl.kernel,
        mesh=_mesh(),
        out_type=jax.ShapeDtypeStruct((vocab // 2, 2 * dim), jnp.float32),
        scratch_types=[
            [pltpu.VMEM((dim, _CC), jnp.float32) for _ in range(2)],
            [pltpu.VMEM((rows_per_u, 2 * dim), jnp.float32) for _ in range(2)],
            pltpu.VMEM((dim, max(tail, 8)), jnp.float32),
            pltpu.VMEM((max(tail // 2, 4), 2 * dim), jnp.float32),
            [pltpu.SemaphoreType.DMA for _ in range(2)],
            [pltpu.SemaphoreType.DMA for _ in range(2)],
        ],
        compiler_params=pltpu.CompilerParams(needs_layout_passes=False),
    )
    def repack_kernel(tt_hbm, packed_hbm, blk, bt, blkt_v, btt_v, lsem, ssem):
        wid = lax.axis_index("s") * _NUM_CORES + lax.axis_index("c")

        def unit_of(u):
            return wid + u * _NUM_WORKERS

        def load_start(u, b):
            @pl.when(unit_of(u) < n_full)
            def _():
                pltpu.async_copy(
                    tt_hbm.at[:, pl.ds(unit_of(u) * _CC, _CC)], blk[b],
                    lsem[b])

        def load_wait(b):
            pltpu.make_async_copy(
                tt_hbm.at[:, pl.ds(0, _CC)], blk[b], lsem[b]).wait()

        def store_start(u, b):
            pltpu.async_copy(
                bt[b], packed_hbm.at[pl.ds(unit_of(u) * rows_per_u,
                                           rows_per_u)], ssem[b])

        def store_wait(b):
            pltpu.make_async_copy(
                bt[b], packed_hbm.at[pl.ds(0, rows_per_u)], ssem[b]).wait()

        for b in range(2):
            load_start(b, b)

        @pl.loop(0, units_per_w)
        def _unit(u):
            b = lax.rem(u, 2)

            @pl.when(unit_of(u) < n_full)
            def _():
                for bb in range(2):

                    @pl.when(b == bb)
                    def _():
                        load_wait(bb)

                        @pl.when(u >= 2)
                        def _():
                            store_wait(bb)

                        @plsc.parallel_loop(0, _CC, step=_L, unroll=2)
                        def _tp(v0):
                            vv = lax.iota(jnp.int32, _L) + v0
                            rvec = lax.shift_right_logical(vv, 1)
                            cbase = lax.mul(lax.bitwise_and(vv, 1), dim)
                            for d in range(dim):
                                vals = blk[bb][d, pl.ds(v0, _L)]
                                plsc.store_scatter(
                                    bt[bb], [rvec, cbase + d], vals)

                        store_start(u, bb)
                        load_start(u + 2, bb)

        for b in range(2):

            @pl.when(unit_of(units_per_w - 2 + b) < n_full)
            def _():
                store_wait(b)

        if tail:

            @pl.when(wid == _NUM_WORKERS - 1)
            def _():
                v0 = n_full * _CC
                pltpu.sync_copy(tt_hbm.at[:, pl.ds(v0, tail)], blkt_v)

                @plsc.parallel_loop(0, tail, step=_L, unroll=2)
                def _tpt(v0):
                    vv = lax.iota(jnp.int32, _L) + v0
                    rvec = lax.shift_right_logical(vv, 1)
                    cbase = lax.mul(lax.bitwise_and(vv, 1), dim)
                    for d in range(dim):
                        vals = blkt_v[d, pl.ds(v0, _L)]
                        plsc.store_scatter(btt_v, [rvec, cbase + d], vals)

                pltpu.sync_copy(
                    btt_v, packed_hbm.at[pl.ds(v0 // 2, tail // 2)])

    return repack_kernel


@functools.lru_cache(maxsize=None)
def _make_gather(batch: int, hist: int, vocab: int, dim: int):
    assert dim == 64 and vocab % 2 == 0 and batch % _CB == 0
    n_units = hist * (batch // _CB)
    assert n_units % _NUM_WORKERS == 0
    units_per_w = n_units // _NUM_WORKERS
    assert units_per_w >= 2
    chunks_per_h = batch // _CB

    @functools.partial(
        pl.kernel,
        mesh=_mesh(),
        out_type=jax.ShapeDtypeStruct((hist, dim, batch), jnp.float32),
        scratch_types=[
            [pltpu.VMEM((_CB,), jnp.int32) for _ in range(2)],
            [pltpu.VMEM((_CB,), jnp.int32) for _ in range(2)],
            [pltpu.VMEM((_CB, 2 * dim), jnp.float32) for _ in range(2)],
            [pltpu.VMEM((dim, _CB), jnp.float32) for _ in range(2)],
            [pltpu.SemaphoreType.DMA for _ in range(2)],
            [pltpu.SemaphoreType.DMA for _ in range(2)],
        ],
        compiler_params=pltpu.CompilerParams(needs_layout_passes=False),
    )
    def gather_kernel(idx_hbm, table_hbm, out_hbm, idx, row, rows, qt,
                      gsem, ssem):
        wid = lax.axis_index("s") * _NUM_CORES + lax.axis_index("c")
        ubase = wid * units_per_w

        def fetch(u, b):
            unit = ubase + u
            h = unit // chunks_per_h
            b0 = (unit % chunks_per_h) * _CB
            pltpu.sync_copy(idx_hbm.at[pl.ds(h * batch + b0, _CB)], idx[b])

            @plsc.parallel_loop(0, _CB, step=_L, unroll=4)
            def _rows(j):
                v = idx[b][pl.ds(j, _L)]
                row[b][pl.ds(j, _L)] = lax.shift_right_logical(v, 1)

            pltpu.async_copy(table_hbm.at[row[b]], rows[b], gsem[b])

        def gather_wait(b):
            pltpu.make_async_copy(
                table_hbm.at[row[b]], rows[b], gsem[b]).wait()

        def store_start(u, b):
            unit = ubase + u
            h = unit // chunks_per_h
            b0 = (unit % chunks_per_h) * _CB
            pltpu.async_copy(
                qt[b], out_hbm.at[h, :, pl.ds(b0, _CB)], ssem[b])

        def store_wait(b):
            pltpu.make_async_copy(
                qt[b], out_hbm.at[0, :, pl.ds(0, _CB)], ssem[b]).wait()

        for b in range(2):
            fetch(b, b)

        @pl.loop(0, units_per_w)
        def _unit(u):
            b = lax.rem(u, 2)
            for bb in range(2):

                @pl.when(b == bb)
                def _():
                    gather_wait(bb)

                    @pl.when(u >= 2)
                    def _():
                        store_wait(bb)

                    @plsc.parallel_loop(0, _CB, step=_L, unroll=2)
                    def _tp(j0):
                        vidx = idx[bb][pl.ds(j0, _L)]
                        hvec = lax.mul(lax.bitwise_and(vidx, 1), dim)
                        for i in range(_L):
                            half = hvec[i]
                            jvec = jnp.broadcast_to(j0 + i, (_L,))
                            for d0 in range(0, dim, _L):
                                vals = rows[bb][j0 + i, pl.ds(half + d0, _L)]
                                plsc.store_scatter(
                                    qt[bb],
                                    [lax.iota(jnp.int32, _L) + d0, jvec],
                                    vals)

                    store_start(u, bb)

                    @pl.when(u + 2 < units_per_w)
                    def _():
                        fetch(u + 2, bb)

        for b in range(2):
            store_wait(b)

    return gather_kernel


def kernel(x, table):
    batch, hist = x.shape
    vocab, dim = table.shape
    idx_hm = jnp.transpose(x).reshape(-1).astype(jnp.int32)
    table_packed = _make_repack(vocab, dim)(jnp.transpose(table))
    q = _make_gather(batch, hist, vocab, dim)(idx_hm, table_packed)
    return jnp.transpose(q, (2, 0, 1))


# speedup vs baseline: 1.4470x; 1.4470x over previous
import functools

import jax
import jax.numpy as jnp
from jax import lax
from jax.experimental import pallas as pl
from jax.experimental.pallas import tpu as pltpu
from jax.experimental.pallas import tpu_sc as plsc

_NUM_CORES = 2
_NUM_SUBCORES = 16
_NUM_WORKERS = _NUM_CORES * _NUM_SUBCORES

_CHUNK = 512
_NBUF = 2


@functools.lru_cache(maxsize=None)
def _make_gather(num_idx: int, vocab: int, dim: int):
    assert num_idx % (_NUM_WORKERS * _CHUNK) == 0
    per_w = num_idx // _NUM_WORKERS
    n_chunks = per_w // _CHUNK
    assert n_chunks % _NBUF == 0
    mesh = plsc.VectorSubcoreMesh(
        core_axis_name="c", subcore_axis_name="s",
        num_cores=_NUM_CORES, num_subcores=_NUM_SUBCORES)

    @functools.partial(
        pl.kernel,
        mesh=mesh,
        out_type=jax.ShapeDtypeStruct((num_idx, dim), jnp.float32),
        scratch_types=[
            pltpu.VMEM((per_w,), jnp.int32),
            [pltpu.VMEM((_CHUNK, dim), jnp.float32) for _ in range(_NBUF)],
            [pltpu.SemaphoreType.DMA for _ in range(_NBUF)],
            [pltpu.SemaphoreType.DMA for _ in range(_NBUF)],
        ],
        compiler_params=pltpu.CompilerParams(use_tc_tiling_on_sc=False),
    )
    def gather_kernel(idx_hbm, table_hbm, out_hbm, idx_v, rows, gsem, ssem):
        wid = lax.axis_index("s") * _NUM_CORES + lax.axis_index("c")
        base = wid * per_w
        pltpu.sync_copy(idx_hbm.at[pl.ds(base, per_w)], idx_v)

        def gather_start(b, c):
            pltpu.async_copy(
                table_hbm.at[idx_v.at[pl.ds(c * _CHUNK, _CHUNK)]],
                rows[b], gsem[b])

        def gather_wait(b):
            pltpu.make_async_copy(
                table_hbm.at[idx_v.at[pl.ds(0, _CHUNK)]],
                rows[b], gsem[b]).wait()

        def store_start(b, c):
            pltpu.async_copy(
                rows[b], out_hbm.at[pl.ds(base + c * _CHUNK, _CHUNK)], ssem[b])

        def store_wait(b):
            pltpu.make_async_copy(
                rows[b], out_hbm.at[pl.ds(base, _CHUNK)], ssem[b]).wait()

        for b in range(_NBUF):
            gather_start(b, b)

        @pl.loop(0, n_chunks, step=_NBUF)
        def _group(i):
            for b in range(_NBUF):
                gather_wait(b)
                store_start(b, i + b)
            for b in range(_NBUF):
                store_wait(b)
                nxt = i + _NBUF + b

                @pl.when(nxt < n_chunks)
                def _():
                    gather_start(b, nxt)

    return gather_kernel


def kernel(x, table):
    batch, hist = x.shape
    vocab, dim = table.shape
    flat_idx = x.reshape(-1).astype(jnp.int32)
    out = _make_gather(flat_idx.shape[0], vocab, dim)(flat_idx, table)
    return out.reshape(batch, hist, dim)
